# baseline (speedup 1.0000x reference)
import functools

import jax
import jax.numpy as jnp
from jax import lax
from jax.experimental import pallas as pl
from jax.experimental.pallas import tpu as pltpu, tpu_sc as plsc

BATCH = 4096
EMBED = 32

_INFO = plsc.get_sparse_core_info()
_NC, _NS, _L = _INFO.num_cores, _INFO.num_subcores, _INFO.num_lanes
_NW = _NC * _NS
_BPW = BATCH // _NW
_GROUPS = _BPW // _L
_D = 12

_mesh = plsc.VectorSubcoreMesh(core_axis_name="c", subcore_axis_name="s")


@functools.partial(
    pl.kernel,
    mesh=_mesh,
    out_type=jax.ShapeDtypeStruct((BATCH,), jnp.float32),
    scratch_types=[
        pltpu.VMEM((_BPW,), jnp.int32),
        pltpu.VMEM((_BPW,), jnp.int32),
        pltpu.VMEM((_D, EMBED, 128), jnp.float32),
        pltpu.VMEM((_D, EMBED, 128), jnp.float32),
        pltpu.VMEM((_BPW * _L,), jnp.float32),
        pltpu.VMEM((_BPW,), jnp.float32),
        pltpu.SemaphoreType.DMA,
        pltpu.SemaphoreType.DMA,
    ],
    compiler_params=pltpu.CompilerParams(
        needs_layout_passes=False, use_tc_tiling_on_sc=True),
)
def _sc_dot(users_hbm, movies_hbm, ut_hbm, mt_hbm, out_hbm,
            uidx_v, midx_v, ubuf, mbuf, hbuf, out_v,
            usem, msem):
    wid = lax.axis_index("s") * _NC + lax.axis_index("c")
    base = wid * _BPW
    pltpu.sync_copy(users_hbm.at[pl.ds(base, _BPW)], uidx_v)
    pltpu.sync_copy(movies_hbm.at[pl.ds(base, _BPW)], midx_v)
    lane = lax.iota(jnp.int32, _L)

    def fetch(tbl, idx_v, buf, sem, r, slot):
        g, s = r // _L, r % _L
        v = idx_v[pl.ds(g * _L, _L)]
        c = jnp.max(jnp.where(lane == s, v, 0))
        c128 = pl.multiple_of((c // 128) * 128, 128)
        return pltpu.async_copy(tbl.at[:, pl.ds(c128, 128)], buf.at[slot], sem)

    ucopies = [fetch(ut_hbm, uidx_v, ubuf, usem, r, r) for r in range(_D)]
    mcopies = [fetch(mt_hbm, midx_v, mbuf, msem, r, r) for r in range(_D)]
    for r in range(_BPW):
        slot = r % _D
        ucopies[slot].wait()
        mcopies[slot].wait()
        g, s = r // _L, r % _L
        slot_v = jnp.full((_L,), slot, jnp.int32)
        sel = jnp.full((_L,), s, jnp.int32)
        cu = jnp.take_along_axis(uidx_v[pl.ds(g * _L, _L)] & 127, sel,
                                 axis=0, mode="promise_in_bounds")
        cm = jnp.take_along_axis(midx_v[pl.ds(g * _L, _L)] & 127, sel,
                                 axis=0, mode="promise_in_bounds")
        u0 = plsc.load_gather(ubuf, [slot_v, lane, cu])
        u1 = plsc.load_gather(ubuf, [slot_v, lane + _L, cu])
        m0 = plsc.load_gather(mbuf, [slot_v, lane, cm])
        m1 = plsc.load_gather(mbuf, [slot_v, lane + _L, cm])
        hbuf[pl.ds(r * _L, _L)] = u0 * m0 + u1 * m1
        if r + _D < _BPW:
            ucopies[slot] = fetch(ut_hbm, uidx_v, ubuf, usem, r + _D, slot)
            mcopies[slot] = fetch(mt_hbm, midx_v, mbuf, msem, r + _D, slot)
    for g in range(_GROUPS):
        idx_base = lane * _L + (g * _L * _L)
        acc = jnp.zeros((_L,), jnp.float32)
        for j in range(_L):
            acc = acc + plsc.load_gather(hbuf, [idx_base + j])
        out_v[pl.ds(g * _L, _L)] = acc
    pltpu.sync_copy(out_v, out_hbm.at[pl.ds(base, _BPW)])


def kernel(users, movies, user_table, movie_table):
    out = _sc_dot(users.astype(jnp.int32), movies.astype(jnp.int32),
                  user_table.T, movie_table.T)
    return out[:, None]


# speedup vs baseline: 1.0469x; 1.0469x over previous
import functools

import jax
import jax.numpy as jnp
from jax import lax
from jax.experimental import pallas as pl
from jax.experimental.pallas import tpu as pltpu, tpu_sc as plsc

BATCH = 4096
EMBED = 32

_INFO = plsc.get_sparse_core_info()
_NC, _NS, _L = _INFO.num_cores, _INFO.num_subcores, _INFO.num_lanes
_NW = _NC * _NS
_BPW = BATCH // _NW
_GROUPS = _BPW // _L
_D = 8

_mesh = plsc.VectorSubcoreMesh(core_axis_name="c", subcore_axis_name="s")


@functools.partial(
    pl.kernel,
    mesh=_mesh,
    out_type=jax.ShapeDtypeStruct((BATCH,), jnp.float32),
    scratch_types=[
        pltpu.VMEM((_BPW,), jnp.int32),
        pltpu.VMEM((_BPW,), jnp.int32),
        pltpu.VMEM((_D, EMBED, 128), jnp.float32),
        pltpu.VMEM((_D, EMBED, 128), jnp.float32),
        pltpu.VMEM((_BPW * _L,), jnp.float32),
        pltpu.VMEM((_BPW,), jnp.float32),
        pltpu.SemaphoreType.DMA,
        pltpu.SemaphoreType.DMA,
    ],
    compiler_params=pltpu.CompilerParams(
        needs_layout_passes=False, use_tc_tiling_on_sc=True),
)
def _sc_dot(users_hbm, movies_hbm, ut_hbm, mt_hbm, out_hbm,
            uidx_v, midx_v, ubuf, mbuf, hbuf, out_v,
            usem, msem):
    wid = lax.axis_index("s") * _NC + lax.axis_index("c")
    base = wid * _BPW
    pltpu.sync_copy(users_hbm.at[pl.ds(base, _BPW)], uidx_v)
    pltpu.sync_copy(movies_hbm.at[pl.ds(base, _BPW)], midx_v)
    lane = lax.iota(jnp.int32, _L)

    def packed(g):
        vu = uidx_v[pl.ds(g * _L, _L)]
        vm = midx_v[pl.ds(g * _L, _L)]
        return ((vu >> 7) << 13) | (vm >> 7)

    def fetch(r, slot):
        g, s = r // _L, r % _L
        p = jnp.max(jnp.where(lane == s, packed(g), 0))
        cu128 = pl.multiple_of((p >> 13) * 128, 128)
        cm128 = pl.multiple_of((p & 8191) * 128, 128)
        return (
            pltpu.async_copy(ut_hbm.at[:, pl.ds(cu128, 128)], ubuf.at[slot],
                             usem),
            pltpu.async_copy(mt_hbm.at[:, pl.ds(cm128, 128)], mbuf.at[slot],
                             msem),
        )

    copies = [fetch(r, r) for r in range(_D)]
    for g in range(_GROUPS):
        cu_all = uidx_v[pl.ds(g * _L, _L)] & 127
        cm_all = midx_v[pl.ds(g * _L, _L)] & 127
        for s in range(_L):
            r = g * _L + s
            slot = r % _D
            cu_cp, cm_cp = copies[slot]
            cu_cp.wait()
            cm_cp.wait()
            slot_v = jnp.full((_L,), slot, jnp.int32)
            sel = jnp.full((_L,), s, jnp.int32)
            cu = jnp.take_along_axis(cu_all, sel, axis=0,
                                     mode="promise_in_bounds")
            cm = jnp.take_along_axis(cm_all, sel, axis=0,
                                     mode="promise_in_bounds")
            u0 = plsc.load_gather(ubuf, [slot_v, lane, cu])
            u1 = plsc.load_gather(ubuf, [slot_v, lane + _L, cu])
            m0 = plsc.load_gather(mbuf, [slot_v, lane, cm])
            m1 = plsc.load_gather(mbuf, [slot_v, lane + _L, cm])
            hbuf[pl.ds(r * _L, _L)] = u0 * m0 + u1 * m1
            if r + _D < _BPW:
                copies[slot] = fetch(r + _D, slot)
    for g in range(_GROUPS):
        idx_base = lane * _L + (g * _L * _L)
        acc = jnp.zeros((_L,), jnp.float32)
        for j in range(_L):
            acc = acc + plsc.load_gather(hbuf, [idx_base + j])
        out_v[pl.ds(g * _L, _L)] = acc
    pltpu.sync_copy(out_v, out_hbm.at[pl.ds(base, _BPW)])


def kernel(users, movies, user_table, movie_table):
    out = _sc_dot(users.astype(jnp.int32), movies.astype(jnp.int32),
                  user_table.T, movie_table.T)
    return out[:, None]
